# baseline (speedup 1.0000x reference)
import functools

import jax
import jax.numpy as jnp
from jax import lax
from jax.experimental import pallas as pl
from jax.experimental.pallas import tpu as pltpu
from jax.experimental.pallas import tpu_sc as plsc

N = 10000
E = 160000
NODE_DIM = 256
EDGE_DIM = 16
GLOBAL_DIM = 64
HIDDEN_DIM = 512
B_GRAPHS = 64

N_TILES = 32
CHUNK = 128
N_CHUNKS = E // CHUNK
COL_ROWS = 1280
BASE_ROWS = N_CHUNKS // N_TILES
MAX_ROWS = BASE_ROWS + 1
ROWS_PER_TILE = 640
SUM_ROWS = 10240

BLK = 1024
GRID = (N + BLK - 1) // BLK


def _sc_scatter_body(ea_hbm, col_hbm, sums_hbm, cnt_hbm,
                     edge_v, idx_v, cnt_v, sums_sh):
    cid = lax.axis_index("c")
    sid = lax.axis_index("s")
    w = cid * 16 + sid
    base = w * BASE_ROWS + jnp.minimum(w, 2)
    nrows = jnp.where(w < 2, MAX_ROWS, BASE_ROWS)
    dma_base = jnp.minimum(base, N_CHUNKS - MAX_ROWS)
    off = base - dma_base

    pltpu.sync_copy(ea_hbm.at[pl.ds(dma_base * CHUNK, MAX_ROWS * CHUNK)],
                    edge_v)
    pltpu.sync_copy(col_hbm.at[pl.ds(dma_base, MAX_ROWS)], idx_v)
    zeros16 = jnp.zeros((16,), jnp.float32)

    def zero_step(i, carry):
        cnt_v[i] = zeros16
        return carry

    lax.fori_loop(0, ROWS_PER_TILE, zero_step, 0)
    pltpu.sync_copy(cnt_v, sums_sh.at[pl.ds(sid * ROWS_PER_TILE,
                                            ROWS_PER_TILE)])
    plsc.subcore_barrier()

    ones = jnp.full((16,), 1.0, jnp.float32)

    def chunk_step(j, carry):
        row = off + j
        pltpu.sync_copy(edge_v.at[pl.ds(row * CHUNK, CHUNK)],
                        sums_sh.at[idx_v.at[row]], add=True)
        for k in range(CHUNK // 16):
            c = idx_v[row, pl.ds(k * 16, 16)]
            plsc.addupdate_scatter(
                cnt_v, [lax.shift_right_logical(c, 4),
                        lax.bitwise_and(c, 15)], ones)
        return carry

    lax.fori_loop(0, nrows, chunk_step, 0)
    plsc.subcore_barrier()

    pltpu.sync_copy(sums_sh.at[pl.ds(sid * ROWS_PER_TILE, ROWS_PER_TILE)],
                    sums_hbm.at[cid, pl.ds(sid * ROWS_PER_TILE,
                                           ROWS_PER_TILE)])
    pltpu.sync_copy(cnt_v, cnt_hbm.at[w])


@functools.cache
def _get_sc_scatter():
    return functools.partial(
        pl.kernel,
        out_type=[
            jax.ShapeDtypeStruct((2, SUM_ROWS, EDGE_DIM), jnp.float32),
            jax.ShapeDtypeStruct((N_TILES, ROWS_PER_TILE, EDGE_DIM),
                                 jnp.float32),
        ],
        mesh=plsc.VectorSubcoreMesh(core_axis_name="c", subcore_axis_name="s",
                                    num_cores=2, num_subcores=16),
        scratch_types=[
            pltpu.VMEM((MAX_ROWS * CHUNK, EDGE_DIM), jnp.float32),
            pltpu.VMEM((MAX_ROWS, CHUNK), jnp.int32),
            pltpu.VMEM((ROWS_PER_TILE, EDGE_DIM), jnp.float32),
            pltpu.VMEM_SHARED((SUM_ROWS, EDGE_DIM), jnp.float32),
        ],
        compiler_params=pltpu.CompilerParams(needs_layout_passes=False,
                                             use_tc_tiling_on_sc=False),
    )(_sc_scatter_body)


def _tc_mlp_body(x_ref, s_ref, cnt_ref, b_ref, u_ref,
                 w1x_ref, w1a_ref, w1u_ref, b1_ref, w2_ref, b2_ref, o_ref):
    ones = jnp.ones((N_TILES, 1), jnp.float32)
    c = lax.dot_general(cnt_ref[...], ones, (((0,), (0,)), ((), ())),
                        preferred_element_type=jnp.float32)
    inv = 1.0 / jnp.maximum(c, 1.0)
    oh = (b_ref[...] == lax.broadcasted_iota(jnp.int32, (BLK, B_GRAPHS), 1)
          ).astype(jnp.float32)
    uw = jnp.dot(u_ref[...], w1u_ref[...], preferred_element_type=jnp.float32)
    acc = jnp.dot(x_ref[...], w1x_ref[...], preferred_element_type=jnp.float32)
    acc = acc + jnp.dot(s_ref[0] + s_ref[1], w1a_ref[...],
                        preferred_element_type=jnp.float32) * inv
    acc = acc + jnp.dot(oh, uw, preferred_element_type=jnp.float32)
    h1 = jnp.maximum(acc + b1_ref[...], 0.0)
    o_ref[...] = (jnp.dot(h1, w2_ref[...], preferred_element_type=jnp.float32)
                  + b2_ref[...])


def _tc_mlp(x, s, cnt, batch2d, u, w1x, w1a, w1u, b1r, w2, b2r):
    return pl.pallas_call(
        _tc_mlp_body,
        grid=(GRID,),
        in_specs=[
            pl.BlockSpec((BLK, NODE_DIM), lambda i: (i, 0)),
            pl.BlockSpec((2, BLK, EDGE_DIM), lambda i: (0, i, 0)),
            pl.BlockSpec((N_TILES, BLK), lambda i: (0, i)),
            pl.BlockSpec((BLK, 1), lambda i: (i, 0)),
            pl.BlockSpec((B_GRAPHS, GLOBAL_DIM), lambda i: (0, 0)),
            pl.BlockSpec((NODE_DIM, HIDDEN_DIM), lambda i: (0, 0)),
            pl.BlockSpec((EDGE_DIM, HIDDEN_DIM), lambda i: (0, 0)),
            pl.BlockSpec((GLOBAL_DIM, HIDDEN_DIM), lambda i: (0, 0)),
            pl.BlockSpec((1, HIDDEN_DIM), lambda i: (0, 0)),
            pl.BlockSpec((HIDDEN_DIM, NODE_DIM), lambda i: (0, 0)),
            pl.BlockSpec((1, NODE_DIM), lambda i: (0, 0)),
        ],
        out_specs=pl.BlockSpec((BLK, NODE_DIM), lambda i: (i, 0)),
        out_shape=jax.ShapeDtypeStruct((N, NODE_DIM), jnp.float32),
        compiler_params=pltpu.CompilerParams(
            dimension_semantics=("arbitrary",)),
    )(x, s, cnt, batch2d, u, w1x, w1a, w1u, b1r, w2, b2r)


def kernel(x, edge_index, edge_attr, u, batch, W1, b1, W2, b2):
    col = edge_index[1].astype(jnp.int32)
    col2d = jnp.concatenate(
        [col, jnp.zeros(((COL_ROWS - N_CHUNKS) * CHUNK,), jnp.int32)]
    ).reshape(COL_ROWS, CHUNK)
    sums, cnt_t = _get_sc_scatter()(edge_attr, col2d)
    cnt = cnt_t.reshape(N_TILES, SUM_ROWS)

    batch2d = batch.astype(jnp.int32).reshape(N, 1)
    w1x = W1[:NODE_DIM]
    w1a = W1[NODE_DIM:NODE_DIM + EDGE_DIM]
    w1u = W1[NODE_DIM + EDGE_DIM:]
    b1r = b1.reshape(1, HIDDEN_DIM)
    b2r = b2.reshape(1, NODE_DIM)
    return _tc_mlp(x, sums, cnt, batch2d, u, w1x, w1a, w1u, b1r, W2, b2r)


# speedup vs baseline: 1.1655x; 1.1655x over previous
import functools

import jax
import jax.numpy as jnp
from jax import lax
from jax.experimental import pallas as pl
from jax.experimental.pallas import tpu as pltpu
from jax.experimental.pallas import tpu_sc as plsc

N = 10000
E = 160000
NODE_DIM = 256
EDGE_DIM = 16
GLOBAL_DIM = 64
HIDDEN_DIM = 512
B_GRAPHS = 64

N_TILES = 32
CHUNK = 128
N_CHUNKS = E // CHUNK
COL_ROWS = 1280
BASE_ROWS = N_CHUNKS // N_TILES
MAX_ROWS = BASE_ROWS + 1
ROWS_PER_TILE = 640
SUM_ROWS = 10240

BLK = 1024
GRID = (N + BLK - 1) // BLK


DEPTH = 8


def _sc_scatter_body(ea_hbm, col_hbm, z_hbm, sums_hbm, cnt_hbm,
                     edge_v, idx_v, cnt_v, sums_sh, sem_in, sem_sc):
    cid = lax.axis_index("c")
    sid = lax.axis_index("s")
    w = cid * 16 + sid
    base = w * BASE_ROWS + jnp.minimum(w, 2)
    nrows = jnp.where(w < 2, MAX_ROWS, BASE_ROWS)
    dma_base = jnp.minimum(base, N_CHUNKS - MAX_ROWS)
    off = base - dma_base

    in0 = pltpu.async_copy(
        ea_hbm.at[pl.ds(dma_base * CHUNK, MAX_ROWS * CHUNK)], edge_v, sem_in)
    in1 = pltpu.async_copy(col_hbm.at[pl.ds(dma_base, MAX_ROWS)], idx_v,
                           sem_in)
    in2 = pltpu.async_copy(
        z_hbm, sums_sh.at[pl.ds(sid * ROWS_PER_TILE, ROWS_PER_TILE)], sem_in)
    zeros16 = jnp.zeros((16,), jnp.float32)

    def zero_step(i, carry):
        cnt_v[pl.ds(i * 16, 16)] = zeros16
        return carry

    lax.fori_loop(0, SUM_ROWS // 16, zero_step, 0)
    in0.wait()
    in1.wait()
    in2.wait()
    plsc.subcore_barrier()

    ones = jnp.full((16,), 1.0, jnp.float32)

    def scatter_of(j):
        row = off + j
        return pltpu.make_async_copy(edge_v.at[pl.ds(row * CHUNK, CHUNK)],
                                     sums_sh.at[idx_v.at[row]], sem_sc)

    def chunk_step(j, carry):
        row = off + j
        scatter_of(j).start(add=True)

        @pl.when(j >= DEPTH)
        def _():
            scatter_of(j - DEPTH).wait()

        for k in range(CHUNK // 16):
            c = idx_v[row, pl.ds(k * 16, 16)]
            plsc.addupdate_scatter(cnt_v, [c], ones)
        return carry

    lax.fori_loop(0, nrows, chunk_step, 0)

    def drain_step(j, carry):
        scatter_of(j).wait()
        return carry

    lax.fori_loop(nrows - DEPTH, nrows, drain_step, 0)
    plsc.subcore_barrier()

    pltpu.sync_copy(sums_sh.at[pl.ds(sid * ROWS_PER_TILE, ROWS_PER_TILE)],
                    sums_hbm.at[cid, pl.ds(sid * ROWS_PER_TILE,
                                           ROWS_PER_TILE)])
    pltpu.sync_copy(cnt_v, cnt_hbm.at[w])


@functools.cache
def _get_sc_scatter():
    return functools.partial(
        pl.kernel,
        out_type=[
            jax.ShapeDtypeStruct((2, SUM_ROWS, EDGE_DIM), jnp.float32),
            jax.ShapeDtypeStruct((N_TILES, SUM_ROWS), jnp.float32),
        ],
        mesh=plsc.VectorSubcoreMesh(core_axis_name="c", subcore_axis_name="s",
                                    num_cores=2, num_subcores=16),
        scratch_types=[
            pltpu.VMEM((MAX_ROWS * CHUNK, EDGE_DIM), jnp.float32),
            pltpu.VMEM((MAX_ROWS, CHUNK), jnp.int32),
            pltpu.VMEM((SUM_ROWS,), jnp.float32),
            pltpu.VMEM_SHARED((SUM_ROWS, EDGE_DIM), jnp.float32),
            pltpu.SemaphoreType.DMA,
            pltpu.SemaphoreType.DMA,
        ],
        compiler_params=pltpu.CompilerParams(needs_layout_passes=False,
                                             use_tc_tiling_on_sc=False),
    )(_sc_scatter_body)


def _tc_mlp_body(x_ref, s_ref, cnt_ref, b_ref, u_ref,
                 w1x_ref, w1a_ref, w1u_ref, b1_ref, w2_ref, b2_ref, o_ref):
    ones = jnp.ones((N_TILES, 1), jnp.float32)
    c = lax.dot_general(cnt_ref[...], ones, (((0,), (0,)), ((), ())),
                        preferred_element_type=jnp.float32)
    inv = 1.0 / jnp.maximum(c, 1.0)
    oh = (b_ref[...] == lax.broadcasted_iota(jnp.int32, (BLK, B_GRAPHS), 1)
          ).astype(jnp.float32)
    uw = jnp.dot(u_ref[...], w1u_ref[...], preferred_element_type=jnp.float32)
    acc = jnp.dot(x_ref[...], w1x_ref[...], preferred_element_type=jnp.float32)
    acc = acc + jnp.dot(s_ref[0] + s_ref[1], w1a_ref[...],
                        preferred_element_type=jnp.float32) * inv
    acc = acc + jnp.dot(oh, uw, preferred_element_type=jnp.float32)
    h1 = jnp.maximum(acc + b1_ref[...], 0.0)
    o_ref[...] = (jnp.dot(h1, w2_ref[...], preferred_element_type=jnp.float32)
                  + b2_ref[...])


def _tc_mlp(x, s, cnt, batch2d, u, w1x, w1a, w1u, b1r, w2, b2r):
    return pl.pallas_call(
        _tc_mlp_body,
        grid=(GRID,),
        in_specs=[
            pl.BlockSpec((BLK, NODE_DIM), lambda i: (i, 0)),
            pl.BlockSpec((2, BLK, EDGE_DIM), lambda i: (0, i, 0)),
            pl.BlockSpec((N_TILES, BLK), lambda i: (0, i)),
            pl.BlockSpec((BLK, 1), lambda i: (i, 0)),
            pl.BlockSpec((B_GRAPHS, GLOBAL_DIM), lambda i: (0, 0)),
            pl.BlockSpec((NODE_DIM, HIDDEN_DIM), lambda i: (0, 0)),
            pl.BlockSpec((EDGE_DIM, HIDDEN_DIM), lambda i: (0, 0)),
            pl.BlockSpec((GLOBAL_DIM, HIDDEN_DIM), lambda i: (0, 0)),
            pl.BlockSpec((1, HIDDEN_DIM), lambda i: (0, 0)),
            pl.BlockSpec((HIDDEN_DIM, NODE_DIM), lambda i: (0, 0)),
            pl.BlockSpec((1, NODE_DIM), lambda i: (0, 0)),
        ],
        out_specs=pl.BlockSpec((BLK, NODE_DIM), lambda i: (i, 0)),
        out_shape=jax.ShapeDtypeStruct((N, NODE_DIM), jnp.float32),
        compiler_params=pltpu.CompilerParams(
            dimension_semantics=("arbitrary",)),
    )(x, s, cnt, batch2d, u, w1x, w1a, w1u, b1r, w2, b2r)


def kernel(x, edge_index, edge_attr, u, batch, W1, b1, W2, b2):
    col = edge_index[1].astype(jnp.int32)
    col2d = jnp.concatenate(
        [col, jnp.zeros(((COL_ROWS - N_CHUNKS) * CHUNK,), jnp.int32)]
    ).reshape(COL_ROWS, CHUNK)
    z = jnp.zeros((ROWS_PER_TILE, EDGE_DIM), jnp.float32)

    sums, cnt = _get_sc_scatter()(edge_attr, col2d, z)

    batch2d = batch.astype(jnp.int32).reshape(N, 1)
    w1x = W1[:NODE_DIM]
    w1a = W1[NODE_DIM:NODE_DIM + EDGE_DIM]
    w1u = W1[NODE_DIM + EDGE_DIM:]
    b1r = b1.reshape(1, HIDDEN_DIM)
    b2r = b2.reshape(1, NODE_DIM)
    return _tc_mlp(x, sums, cnt, batch2d, u, w1x, w1a, w1u, b1r, W2, b2r)


# speedup vs baseline: 1.1880x; 1.0192x over previous
import functools

import jax
import jax.numpy as jnp
from jax import lax
from jax.experimental import pallas as pl
from jax.experimental.pallas import tpu as pltpu
from jax.experimental.pallas import tpu_sc as plsc

N = 10000
E = 160000
NODE_DIM = 256
EDGE_DIM = 16
GLOBAL_DIM = 64
HIDDEN_DIM = 512
B_GRAPHS = 64

N_TILES = 32
CHUNK = 128
N_CHUNKS = E // CHUNK
COL_ROWS = 1280
BASE_ROWS = N_CHUNKS // N_TILES
MAX_ROWS = BASE_ROWS + 1
ROWS_PER_TILE = 640
SUM_ROWS = 10240

BLK = 1024
GRID = (N + BLK - 1) // BLK


DEPTH = 8


def _sc_scatter_body(ea_hbm, col_hbm, z_hbm, sums_hbm, cnt_hbm,
                     edge_v, idx_v, cnt_v, sums_sh, sem_in, sem_sc):
    cid = lax.axis_index("c")
    sid = lax.axis_index("s")
    w = cid * 16 + sid
    base = w * BASE_ROWS + jnp.minimum(w, 2)
    nrows = jnp.where(w < 2, MAX_ROWS, BASE_ROWS)
    dma_base = jnp.minimum(base, N_CHUNKS - MAX_ROWS)
    off = base - dma_base

    in0 = pltpu.async_copy(
        ea_hbm.at[pl.ds(dma_base * CHUNK, MAX_ROWS * CHUNK)], edge_v, sem_in)
    in1 = pltpu.async_copy(col_hbm.at[pl.ds(dma_base, MAX_ROWS)], idx_v,
                           sem_in)
    in2 = pltpu.async_copy(
        z_hbm, sums_sh.at[pl.ds(sid * ROWS_PER_TILE, ROWS_PER_TILE)], sem_in)
    zeros16 = jnp.zeros((16,), jnp.float32)

    def zero_step(i, carry):
        cnt_v[pl.ds(i * 16, 16)] = zeros16
        return carry

    lax.fori_loop(0, SUM_ROWS // 16, zero_step, 0)
    in0.wait()
    in1.wait()
    in2.wait()
    plsc.subcore_barrier()

    ones = jnp.full((16,), 1.0, jnp.float32)

    def scatter_of(j):
        row = off + j
        return pltpu.make_async_copy(edge_v.at[pl.ds(row * CHUNK, CHUNK)],
                                     sums_sh.at[idx_v.at[row]], sem_sc)

    def chunk_step(j, carry):
        row = off + j
        scatter_of(j).start(add=True)

        @pl.when(j >= DEPTH)
        def _():
            scatter_of(j - DEPTH).wait()

        for k in range(CHUNK // 16):
            c = idx_v[row, pl.ds(k * 16, 16)]
            plsc.addupdate_scatter(cnt_v, [c], ones)
        return carry

    lax.fori_loop(0, nrows, chunk_step, 0)

    def drain_step(j, carry):
        scatter_of(j).wait()
        return carry

    lax.fori_loop(nrows - DEPTH, nrows, drain_step, 0)
    plsc.subcore_barrier()

    pltpu.sync_copy(sums_sh.at[pl.ds(sid * ROWS_PER_TILE, ROWS_PER_TILE)],
                    sums_hbm.at[cid, pl.ds(sid * ROWS_PER_TILE,
                                           ROWS_PER_TILE)])
    pltpu.sync_copy(cnt_v, cnt_hbm.at[w])


@functools.cache
def _get_sc_scatter():
    return functools.partial(
        pl.kernel,
        out_type=[
            jax.ShapeDtypeStruct((2, SUM_ROWS, EDGE_DIM), jnp.float32),
            jax.ShapeDtypeStruct((N_TILES, SUM_ROWS), jnp.float32),
        ],
        mesh=plsc.VectorSubcoreMesh(core_axis_name="c", subcore_axis_name="s",
                                    num_cores=2, num_subcores=16),
        scratch_types=[
            pltpu.VMEM((MAX_ROWS * CHUNK, EDGE_DIM), jnp.float32),
            pltpu.VMEM((MAX_ROWS, CHUNK), jnp.int32),
            pltpu.VMEM((SUM_ROWS,), jnp.float32),
            pltpu.VMEM_SHARED((SUM_ROWS, EDGE_DIM), jnp.float32),
            pltpu.SemaphoreType.DMA,
            pltpu.SemaphoreType.DMA,
        ],
        compiler_params=pltpu.CompilerParams(needs_layout_passes=False,
                                             use_tc_tiling_on_sc=False),
    )(_sc_scatter_body)


def _tc_mlp_body(x_ref, s_ref, cnt_ref, b_ref, u_ref,
                 w1x_ref, w1a_ref, w1u_ref, b1_ref, w2_ref, b2_ref, o_ref):
    ones = jnp.ones((N_TILES, 1), jnp.float32)
    c = lax.dot_general(cnt_ref[...], ones, (((0,), (0,)), ((), ())),
                        preferred_element_type=jnp.float32)
    inv = 1.0 / jnp.maximum(c, 1.0)
    oh = (b_ref[...].astype(jnp.int32)
          == lax.broadcasted_iota(jnp.int32, (BLK, B_GRAPHS), 1)
          ).astype(jnp.float32)
    uw = jnp.dot(u_ref[...], w1u_ref[...], preferred_element_type=jnp.float32)
    acc = jnp.dot(x_ref[...], w1x_ref[...], preferred_element_type=jnp.float32)
    acc = acc + jnp.dot(s_ref[0] + s_ref[1], w1a_ref[...],
                        preferred_element_type=jnp.float32) * inv
    acc = acc + jnp.dot(oh, uw, preferred_element_type=jnp.float32)
    h1 = jnp.maximum(acc + b1_ref[...], 0.0)
    o_ref[...] = (jnp.dot(h1, w2_ref[...], preferred_element_type=jnp.float32)
                  + b2_ref[...])


def _tc_mlp(x, s, cnt, batch2d, u, w1x, w1a, w1u, b1r, w2, b2r):
    return pl.pallas_call(
        _tc_mlp_body,
        grid=(GRID,),
        in_specs=[
            pl.BlockSpec((BLK, NODE_DIM), lambda i: (i, 0)),
            pl.BlockSpec((2, BLK, EDGE_DIM), lambda i: (0, i, 0)),
            pl.BlockSpec((N_TILES, BLK), lambda i: (0, i)),
            pl.BlockSpec((BLK, 1), lambda i: (i, 0)),
            pl.BlockSpec((B_GRAPHS, GLOBAL_DIM), lambda i: (0, 0)),
            pl.BlockSpec((NODE_DIM, HIDDEN_DIM), lambda i: (0, 0)),
            pl.BlockSpec((EDGE_DIM, HIDDEN_DIM), lambda i: (0, 0)),
            pl.BlockSpec((GLOBAL_DIM, HIDDEN_DIM), lambda i: (0, 0)),
            pl.BlockSpec((1, HIDDEN_DIM), lambda i: (0, 0)),
            pl.BlockSpec((HIDDEN_DIM, NODE_DIM), lambda i: (0, 0)),
            pl.BlockSpec((1, NODE_DIM), lambda i: (0, 0)),
        ],
        out_specs=pl.BlockSpec((BLK, NODE_DIM), lambda i: (i, 0)),
        out_shape=jax.ShapeDtypeStruct((N, NODE_DIM), jnp.float32),
        compiler_params=pltpu.CompilerParams(
            dimension_semantics=("arbitrary",)),
    )(x, s, cnt, batch2d, u, w1x, w1a, w1u, b1r, w2, b2r)


def kernel(x, edge_index, edge_attr, u, batch, W1, b1, W2, b2):
    col = edge_index[1].astype(jnp.int32)
    col2d = jnp.concatenate(
        [col, jnp.zeros(((COL_ROWS - N_CHUNKS) * CHUNK,), jnp.int32)]
    ).reshape(COL_ROWS, CHUNK)
    z = jnp.zeros((ROWS_PER_TILE, EDGE_DIM), jnp.float32)

    sums, cnt = _get_sc_scatter()(edge_attr, col2d, z)
    sums_bf = sums.astype(jnp.bfloat16)

    batch2d = batch.astype(jnp.int8).reshape(N, 1)
    w1x = W1[:NODE_DIM]
    w1a = W1[NODE_DIM:NODE_DIM + EDGE_DIM].astype(jnp.bfloat16)
    w1u = W1[NODE_DIM + EDGE_DIM:]
    b1r = b1.reshape(1, HIDDEN_DIM)
    b2r = b2.reshape(1, NODE_DIM)
    return _tc_mlp(x, sums_bf, cnt, batch2d, u, w1x, w1a, w1u, b1r, W2, b2r)


# speedup vs baseline: 1.1910x; 1.0026x over previous
import functools

import jax
import jax.numpy as jnp
from jax import lax
from jax.experimental import pallas as pl
from jax.experimental.pallas import tpu as pltpu
from jax.experimental.pallas import tpu_sc as plsc

N = 10000
E = 160000
NODE_DIM = 256
EDGE_DIM = 16
GLOBAL_DIM = 64
HIDDEN_DIM = 512
B_GRAPHS = 64

N_TILES = 32
CHUNK = 128
N_CHUNKS = E // CHUNK
COL_ROWS = 1280
BASE_ROWS = N_CHUNKS // N_TILES
MAX_ROWS = BASE_ROWS + 1
ROWS_PER_TILE = 640
SUM_ROWS = 10240

BLK = 1024
GRID = (N + BLK - 1) // BLK


DEPTH = 8


def _sc_scatter_body(ea_hbm, col_hbm, z_hbm, sums_hbm, cnt_hbm,
                     edge_v, idx_v, cnt_v, sums_sh, sem_in, sem_sc):
    cid = lax.axis_index("c")
    sid = lax.axis_index("s")
    w = cid * 16 + sid
    base = w * BASE_ROWS + jnp.minimum(w, 2)
    nrows = jnp.where(w < 2, MAX_ROWS, BASE_ROWS)
    dma_base = jnp.minimum(base, N_CHUNKS - MAX_ROWS)
    off = base - dma_base

    in0 = pltpu.async_copy(
        ea_hbm.at[pl.ds(dma_base * CHUNK, MAX_ROWS * CHUNK)], edge_v, sem_in)
    in1 = pltpu.async_copy(col_hbm.at[pl.ds(dma_base, MAX_ROWS)], idx_v,
                           sem_in)
    in2 = pltpu.async_copy(
        z_hbm, sums_sh.at[pl.ds(sid * ROWS_PER_TILE, ROWS_PER_TILE)], sem_in)
    zeros16 = jnp.zeros((16,), jnp.float32)

    def zero_step(i, carry):
        cnt_v[pl.ds(i * 16, 16)] = zeros16
        return carry

    lax.fori_loop(0, SUM_ROWS // 16, zero_step, 0)
    in0.wait()
    in1.wait()
    in2.wait()
    plsc.subcore_barrier()

    ones = jnp.full((16,), 1.0, jnp.float32)

    def scatter_of(j):
        row = off + j
        return pltpu.make_async_copy(edge_v.at[pl.ds(row * CHUNK, CHUNK)],
                                     sums_sh.at[idx_v.at[row]], sem_sc)

    def chunk_step(j, carry):
        row = off + j
        scatter_of(j).start(add=True)

        @pl.when(j >= DEPTH)
        def _():
            scatter_of(j - DEPTH).wait()

        for k in range(CHUNK // 16):
            c = idx_v[row, pl.ds(k * 16, 16)]
            plsc.addupdate_scatter(cnt_v, [c], ones)
        return carry

    lax.fori_loop(0, nrows, chunk_step, 0)

    def drain_step(j, carry):
        scatter_of(j).wait()
        return carry

    lax.fori_loop(nrows - DEPTH, nrows, drain_step, 0)
    plsc.subcore_barrier()

    pltpu.sync_copy(sums_sh.at[pl.ds(sid * ROWS_PER_TILE, ROWS_PER_TILE)],
                    sums_hbm.at[cid, pl.ds(sid * ROWS_PER_TILE,
                                           ROWS_PER_TILE)])
    pltpu.sync_copy(cnt_v, cnt_hbm.at[w])


@functools.cache
def _get_sc_scatter():
    return functools.partial(
        pl.kernel,
        out_type=[
            jax.ShapeDtypeStruct((2, SUM_ROWS, EDGE_DIM), jnp.float32),
            jax.ShapeDtypeStruct((N_TILES, SUM_ROWS), jnp.float32),
        ],
        mesh=plsc.VectorSubcoreMesh(core_axis_name="c", subcore_axis_name="s",
                                    num_cores=2, num_subcores=16),
        scratch_types=[
            pltpu.VMEM((MAX_ROWS * CHUNK, EDGE_DIM), jnp.float32),
            pltpu.VMEM((MAX_ROWS, CHUNK), jnp.int32),
            pltpu.VMEM((SUM_ROWS,), jnp.float32),
            pltpu.VMEM_SHARED((SUM_ROWS, EDGE_DIM), jnp.float32),
            pltpu.SemaphoreType.DMA,
            pltpu.SemaphoreType.DMA,
        ],
        compiler_params=pltpu.CompilerParams(needs_layout_passes=False,
                                             use_tc_tiling_on_sc=False),
    )(_sc_scatter_body)


def _tc_mlp_body(x_ref, s_ref, cnt_ref, b_ref, u_ref,
                 w1x_ref, w1a_ref, w1u_ref, b1_ref, w2_ref, b2_ref, o_ref):
    ones = jnp.ones((N_TILES, 1), jnp.float32)
    c = lax.dot_general(cnt_ref[...], ones, (((0,), (0,)), ((), ())),
                        preferred_element_type=jnp.float32)
    inv = 1.0 / jnp.maximum(c, 1.0)
    oh = (b_ref[...].astype(jnp.int32)
          == lax.broadcasted_iota(jnp.int32, (BLK, B_GRAPHS), 1)
          ).astype(jnp.float32)
    uw = jnp.dot(u_ref[...], w1u_ref[...], preferred_element_type=jnp.float32)
    acc = jnp.dot(x_ref[...].astype(jnp.bfloat16), w1x_ref[...],
                  preferred_element_type=jnp.float32)
    acc = acc + jnp.dot(s_ref[0] + s_ref[1], w1a_ref[...],
                        preferred_element_type=jnp.float32) * inv
    acc = acc + jnp.dot(oh, uw, preferred_element_type=jnp.float32)
    h1 = jnp.maximum(acc + b1_ref[...], 0.0)
    o_ref[...] = (jnp.dot(h1.astype(jnp.bfloat16), w2_ref[...],
                          preferred_element_type=jnp.float32) + b2_ref[...])


def _tc_mlp(x, s, cnt, batch2d, u, w1x, w1a, w1u, b1r, w2, b2r):
    return pl.pallas_call(
        _tc_mlp_body,
        grid=(GRID,),
        in_specs=[
            pl.BlockSpec((BLK, NODE_DIM), lambda i: (i, 0)),
            pl.BlockSpec((2, BLK, EDGE_DIM), lambda i: (0, i, 0)),
            pl.BlockSpec((N_TILES, BLK), lambda i: (0, i)),
            pl.BlockSpec((BLK, 1), lambda i: (i, 0)),
            pl.BlockSpec((B_GRAPHS, GLOBAL_DIM), lambda i: (0, 0)),
            pl.BlockSpec((NODE_DIM, HIDDEN_DIM), lambda i: (0, 0)),
            pl.BlockSpec((EDGE_DIM, HIDDEN_DIM), lambda i: (0, 0)),
            pl.BlockSpec((GLOBAL_DIM, HIDDEN_DIM), lambda i: (0, 0)),
            pl.BlockSpec((1, HIDDEN_DIM), lambda i: (0, 0)),
            pl.BlockSpec((HIDDEN_DIM, NODE_DIM), lambda i: (0, 0)),
            pl.BlockSpec((1, NODE_DIM), lambda i: (0, 0)),
        ],
        out_specs=pl.BlockSpec((BLK, NODE_DIM), lambda i: (i, 0)),
        out_shape=jax.ShapeDtypeStruct((N, NODE_DIM), jnp.float32),
        compiler_params=pltpu.CompilerParams(
            dimension_semantics=("arbitrary",)),
    )(x, s, cnt, batch2d, u, w1x, w1a, w1u, b1r, w2, b2r)


def kernel(x, edge_index, edge_attr, u, batch, W1, b1, W2, b2):
    col = edge_index[1].astype(jnp.int32)
    col2d = jnp.concatenate(
        [col, jnp.zeros(((COL_ROWS - N_CHUNKS) * CHUNK,), jnp.int32)]
    ).reshape(COL_ROWS, CHUNK)
    z = jnp.zeros((ROWS_PER_TILE, EDGE_DIM), jnp.float32)

    sums, cnt = _get_sc_scatter()(edge_attr, col2d, z)
    sums_bf = sums.astype(jnp.bfloat16)

    batch2d = batch.astype(jnp.int8).reshape(N, 1)
    w1x = W1[:NODE_DIM].astype(jnp.bfloat16)
    w1a = W1[NODE_DIM:NODE_DIM + EDGE_DIM].astype(jnp.bfloat16)
    w1u = W1[NODE_DIM + EDGE_DIM:]
    b1r = b1.reshape(1, HIDDEN_DIM)
    b2r = b2.reshape(1, NODE_DIM)
    w2_bf = W2.astype(jnp.bfloat16)
    return _tc_mlp(x, sums_bf, cnt, batch2d, u, w1x, w1a, w1u, b1r, w2_bf,
                   b2r)


# speedup vs baseline: 1.2119x; 1.0175x over previous
import functools

import jax
import jax.numpy as jnp
from jax import lax
from jax.experimental import pallas as pl
from jax.experimental.pallas import tpu as pltpu
from jax.experimental.pallas import tpu_sc as plsc

N = 10000
E = 160000
NODE_DIM = 256
EDGE_DIM = 16
GLOBAL_DIM = 64
HIDDEN_DIM = 512
B_GRAPHS = 64

N_TILES = 32
CHUNK = 128
N_CHUNKS = E // CHUNK
COL_ROWS = 1280
BASE_ROWS = N_CHUNKS // N_TILES
MAX_ROWS = BASE_ROWS + 1
ROWS_PER_TILE = 640
SUM_ROWS = 10240

BLK = 1024
GRID = (N + BLK - 1) // BLK


DEPTH = 8


def _sc_scatter_body(ea_hbm, col_hbm, sums_hbm, cnt_hbm,
                     edge_v, idx_v, cnt_v, zero_v, sums_sh, sem_in, sem_sc):
    cid = lax.axis_index("c")
    sid = lax.axis_index("s")
    w = cid * 16 + sid
    base = w * BASE_ROWS + jnp.minimum(w, 2)
    nrows = jnp.where(w < 2, MAX_ROWS, BASE_ROWS)
    dma_base = jnp.minimum(base, N_CHUNKS - MAX_ROWS)
    off = base - dma_base

    in0 = pltpu.async_copy(
        ea_hbm.at[pl.ds(dma_base * CHUNK, MAX_ROWS * CHUNK)], edge_v, sem_in)
    in1 = pltpu.async_copy(col_hbm.at[pl.ds(dma_base, MAX_ROWS)], idx_v,
                           sem_in)
    zeros16 = jnp.zeros((16,), jnp.float32)

    def zero_step(i, carry):
        cnt_v[pl.ds(i * 16, 16)] = zeros16
        zero_v[i] = zeros16
        return carry

    lax.fori_loop(0, ROWS_PER_TILE, zero_step, 0)
    pltpu.sync_copy(zero_v,
                    sums_sh.at[pl.ds(sid * ROWS_PER_TILE, ROWS_PER_TILE)])
    in0.wait()
    in1.wait()
    plsc.subcore_barrier()

    ones = jnp.full((16,), 1.0, jnp.float32)

    def scatter_of(j):
        row = off + j
        return pltpu.make_async_copy(edge_v.at[pl.ds(row * CHUNK, CHUNK)],
                                     sums_sh.at[idx_v.at[row]], sem_sc)

    def chunk_step(j, carry):
        row = off + j
        scatter_of(j).start(add=True)

        @pl.when(j >= DEPTH)
        def _():
            scatter_of(j - DEPTH).wait()

        for k in range(CHUNK // 16):
            c = idx_v[row, pl.ds(k * 16, 16)]
            plsc.addupdate_scatter(cnt_v, [c], ones)
        return carry

    lax.fori_loop(0, nrows, chunk_step, 0)

    def drain_step(j, carry):
        scatter_of(j).wait()
        return carry

    lax.fori_loop(nrows - DEPTH, nrows, drain_step, 0)
    plsc.subcore_barrier()

    pltpu.sync_copy(sums_sh.at[pl.ds(sid * ROWS_PER_TILE, ROWS_PER_TILE)],
                    sums_hbm.at[cid, pl.ds(sid * ROWS_PER_TILE,
                                           ROWS_PER_TILE)])
    pltpu.sync_copy(cnt_v, cnt_hbm.at[w])


@functools.cache
def _get_sc_scatter():
    return functools.partial(
        pl.kernel,
        out_type=[
            jax.ShapeDtypeStruct((2, SUM_ROWS, EDGE_DIM), jnp.float32),
            jax.ShapeDtypeStruct((N_TILES, SUM_ROWS), jnp.float32),
        ],
        mesh=plsc.VectorSubcoreMesh(core_axis_name="c", subcore_axis_name="s",
                                    num_cores=2, num_subcores=16),
        scratch_types=[
            pltpu.VMEM((MAX_ROWS * CHUNK, EDGE_DIM), jnp.float32),
            pltpu.VMEM((MAX_ROWS, CHUNK), jnp.int32),
            pltpu.VMEM((SUM_ROWS,), jnp.float32),
            pltpu.VMEM((ROWS_PER_TILE, EDGE_DIM), jnp.float32),
            pltpu.VMEM_SHARED((SUM_ROWS, EDGE_DIM), jnp.float32),
            pltpu.SemaphoreType.DMA,
            pltpu.SemaphoreType.DMA,
        ],
        compiler_params=pltpu.CompilerParams(needs_layout_passes=False,
                                             use_tc_tiling_on_sc=False),
    )(_sc_scatter_body)


def _tc_mlp_body(x_ref, s_ref, cnt_ref, b_ref, u_ref,
                 w1x_ref, w1a_ref, w1u_ref, b1_ref, w2_ref, b2_ref, o_ref):
    ones = jnp.ones((N_TILES, 1), jnp.float32)
    c = lax.dot_general(cnt_ref[...], ones, (((0,), (0,)), ((), ())),
                        preferred_element_type=jnp.float32)
    inv = 1.0 / jnp.maximum(c, 1.0)
    oh = (b_ref[...].astype(jnp.int32)
          == lax.broadcasted_iota(jnp.int32, (BLK, B_GRAPHS), 1)
          ).astype(jnp.float32)
    uw = jnp.dot(u_ref[...], w1u_ref[...], preferred_element_type=jnp.float32)
    acc = jnp.dot(x_ref[...].astype(jnp.bfloat16), w1x_ref[...],
                  preferred_element_type=jnp.float32)
    acc = acc + jnp.dot(s_ref[0] + s_ref[1], w1a_ref[...],
                        preferred_element_type=jnp.float32) * inv
    acc = acc + jnp.dot(oh, uw, preferred_element_type=jnp.float32)
    h1 = jnp.maximum(acc + b1_ref[...], 0.0)
    o_ref[...] = (jnp.dot(h1.astype(jnp.bfloat16), w2_ref[...],
                          preferred_element_type=jnp.float32) + b2_ref[...])


def _tc_mlp(x, s, cnt, batch2d, u, w1x, w1a, w1u, b1r, w2, b2r):
    return pl.pallas_call(
        _tc_mlp_body,
        grid=(GRID,),
        in_specs=[
            pl.BlockSpec((BLK, NODE_DIM), lambda i: (i, 0)),
            pl.BlockSpec((2, BLK, EDGE_DIM), lambda i: (0, i, 0)),
            pl.BlockSpec((N_TILES, BLK), lambda i: (0, i)),
            pl.BlockSpec((BLK, 1), lambda i: (i, 0)),
            pl.BlockSpec((B_GRAPHS, GLOBAL_DIM), lambda i: (0, 0)),
            pl.BlockSpec((NODE_DIM, HIDDEN_DIM), lambda i: (0, 0)),
            pl.BlockSpec((EDGE_DIM, HIDDEN_DIM), lambda i: (0, 0)),
            pl.BlockSpec((GLOBAL_DIM, HIDDEN_DIM), lambda i: (0, 0)),
            pl.BlockSpec((1, HIDDEN_DIM), lambda i: (0, 0)),
            pl.BlockSpec((HIDDEN_DIM, NODE_DIM), lambda i: (0, 0)),
            pl.BlockSpec((1, NODE_DIM), lambda i: (0, 0)),
        ],
        out_specs=pl.BlockSpec((BLK, NODE_DIM), lambda i: (i, 0)),
        out_shape=jax.ShapeDtypeStruct((N, NODE_DIM), jnp.float32),
        compiler_params=pltpu.CompilerParams(
            dimension_semantics=("arbitrary",)),
    )(x, s, cnt, batch2d, u, w1x, w1a, w1u, b1r, w2, b2r)


def kernel(x, edge_index, edge_attr, u, batch, W1, b1, W2, b2):
    col = edge_index[1].astype(jnp.int32)
    col2d = jnp.concatenate(
        [col, jnp.zeros(((COL_ROWS - N_CHUNKS) * CHUNK,), jnp.int32)]
    ).reshape(COL_ROWS, CHUNK)
    sums, cnt = _get_sc_scatter()(edge_attr, col2d)
    sums_bf = sums.astype(jnp.bfloat16)

    batch2d = batch.astype(jnp.int8).reshape(N, 1)
    w1x = W1[:NODE_DIM].astype(jnp.bfloat16)
    w1a = W1[NODE_DIM:NODE_DIM + EDGE_DIM].astype(jnp.bfloat16)
    w1u = W1[NODE_DIM + EDGE_DIM:]
    b1r = b1.reshape(1, HIDDEN_DIM)
    b2r = b2.reshape(1, NODE_DIM)
    w2_bf = W2.astype(jnp.bfloat16)
    return _tc_mlp(x, sums_bf, cnt, batch2d, u, w1x, w1a, w1u, b1r, w2_bf,
                   b2r)


# speedup vs baseline: 1.2357x; 1.0196x over previous
import functools

import jax
import jax.numpy as jnp
from jax import lax
from jax.experimental import pallas as pl
from jax.experimental.pallas import tpu as pltpu
from jax.experimental.pallas import tpu_sc as plsc

N = 10000
E = 160000
NODE_DIM = 256
EDGE_DIM = 16
GLOBAL_DIM = 64
HIDDEN_DIM = 512
B_GRAPHS = 64

N_TILES = 32
CHUNK = 128
N_CHUNKS = E // CHUNK
COL_ROWS = 1280
BASE_ROWS = N_CHUNKS // N_TILES
MAX_ROWS = BASE_ROWS + 1
ROWS_PER_TILE = 640
SUM_ROWS = 10240

BLK = 2048
GRID = (N + BLK - 1) // BLK


DEPTH = 8


def _sc_scatter_body(ea_hbm, col_hbm, sums_hbm, cnt_hbm,
                     edge_v, idx_v, cnt_v, zero_v, sums_sh, sem_in, sem_sc):
    cid = lax.axis_index("c")
    sid = lax.axis_index("s")
    w = cid * 16 + sid
    base = w * BASE_ROWS + jnp.minimum(w, 2)
    nrows = jnp.where(w < 2, MAX_ROWS, BASE_ROWS)
    dma_base = jnp.minimum(base, N_CHUNKS - MAX_ROWS)
    off = base - dma_base

    in0 = pltpu.async_copy(
        ea_hbm.at[pl.ds(dma_base * CHUNK, MAX_ROWS * CHUNK)], edge_v, sem_in)
    in1 = pltpu.async_copy(col_hbm.at[pl.ds(dma_base, MAX_ROWS)], idx_v,
                           sem_in)
    zeros16 = jnp.zeros((16,), jnp.float32)

    def zero_step(i, carry):
        cnt_v[pl.ds(i * 16, 16)] = zeros16
        zero_v[i] = zeros16
        return carry

    lax.fori_loop(0, ROWS_PER_TILE, zero_step, 0)
    pltpu.sync_copy(zero_v,
                    sums_sh.at[pl.ds(sid * ROWS_PER_TILE, ROWS_PER_TILE)])
    in0.wait()
    in1.wait()
    plsc.subcore_barrier()

    ones = jnp.full((16,), 1.0, jnp.float32)

    def scatter_of(j):
        row = off + j
        return pltpu.make_async_copy(edge_v.at[pl.ds(row * CHUNK, CHUNK)],
                                     sums_sh.at[idx_v.at[row]], sem_sc)

    def chunk_step(j, carry):
        row = off + j
        scatter_of(j).start(add=True)

        @pl.when(j >= DEPTH)
        def _():
            scatter_of(j - DEPTH).wait()

        for k in range(CHUNK // 16):
            c = idx_v[row, pl.ds(k * 16, 16)]
            plsc.addupdate_scatter(cnt_v, [c], ones)
        return carry

    lax.fori_loop(0, nrows, chunk_step, 0)

    def drain_step(j, carry):
        scatter_of(j).wait()
        return carry

    lax.fori_loop(nrows - DEPTH, nrows, drain_step, 0)
    plsc.subcore_barrier()

    pltpu.sync_copy(sums_sh.at[pl.ds(sid * ROWS_PER_TILE, ROWS_PER_TILE)],
                    sums_hbm.at[cid, pl.ds(sid * ROWS_PER_TILE,
                                           ROWS_PER_TILE)])
    pltpu.sync_copy(cnt_v, cnt_hbm.at[w])


@functools.cache
def _get_sc_scatter():
    return functools.partial(
        pl.kernel,
        out_type=[
            jax.ShapeDtypeStruct((2, SUM_ROWS, EDGE_DIM), jnp.float32),
            jax.ShapeDtypeStruct((N_TILES, SUM_ROWS), jnp.float32),
        ],
        mesh=plsc.VectorSubcoreMesh(core_axis_name="c", subcore_axis_name="s",
                                    num_cores=2, num_subcores=16),
        scratch_types=[
            pltpu.VMEM((MAX_ROWS * CHUNK, EDGE_DIM), jnp.float32),
            pltpu.VMEM((MAX_ROWS, CHUNK), jnp.int32),
            pltpu.VMEM((SUM_ROWS,), jnp.float32),
            pltpu.VMEM((ROWS_PER_TILE, EDGE_DIM), jnp.float32),
            pltpu.VMEM_SHARED((SUM_ROWS, EDGE_DIM), jnp.float32),
            pltpu.SemaphoreType.DMA,
            pltpu.SemaphoreType.DMA,
        ],
        compiler_params=pltpu.CompilerParams(needs_layout_passes=False,
                                             use_tc_tiling_on_sc=False),
    )(_sc_scatter_body)


def _tc_mlp_body(x_ref, s_ref, cnt_ref, b_ref, u_ref,
                 w1x_ref, w1a_ref, w1u_ref, b1_ref, w2_ref, b2_ref, o_ref):
    ones = jnp.ones((N_TILES, 1), jnp.float32)
    c = lax.dot_general(cnt_ref[...], ones, (((0,), (0,)), ((), ())),
                        preferred_element_type=jnp.float32)
    inv = 1.0 / jnp.maximum(c, 1.0)
    oh = (b_ref[...].astype(jnp.int32)
          == lax.broadcasted_iota(jnp.int32, (BLK, B_GRAPHS), 1)
          ).astype(jnp.float32)
    uw = jnp.dot(u_ref[...], w1u_ref[...], preferred_element_type=jnp.float32)
    acc = jnp.dot(x_ref[...].astype(jnp.bfloat16), w1x_ref[...],
                  preferred_element_type=jnp.float32)
    acc = acc + jnp.dot(s_ref[0] + s_ref[1], w1a_ref[...],
                        preferred_element_type=jnp.float32) * inv
    acc = acc + jnp.dot(oh, uw, preferred_element_type=jnp.float32)
    h1 = jnp.maximum(acc + b1_ref[...], 0.0)
    o_ref[...] = (jnp.dot(h1.astype(jnp.bfloat16), w2_ref[...],
                          preferred_element_type=jnp.float32) + b2_ref[...])


def _tc_mlp(x, s, cnt, batch2d, u, w1x, w1a, w1u, b1r, w2, b2r):
    return pl.pallas_call(
        _tc_mlp_body,
        grid=(GRID,),
        in_specs=[
            pl.BlockSpec((BLK, NODE_DIM), lambda i: (i, 0)),
            pl.BlockSpec((2, BLK, EDGE_DIM), lambda i: (0, i, 0)),
            pl.BlockSpec((N_TILES, BLK), lambda i: (0, i)),
            pl.BlockSpec((BLK, 1), lambda i: (i, 0)),
            pl.BlockSpec((B_GRAPHS, GLOBAL_DIM), lambda i: (0, 0)),
            pl.BlockSpec((NODE_DIM, HIDDEN_DIM), lambda i: (0, 0)),
            pl.BlockSpec((EDGE_DIM, HIDDEN_DIM), lambda i: (0, 0)),
            pl.BlockSpec((GLOBAL_DIM, HIDDEN_DIM), lambda i: (0, 0)),
            pl.BlockSpec((1, HIDDEN_DIM), lambda i: (0, 0)),
            pl.BlockSpec((HIDDEN_DIM, NODE_DIM), lambda i: (0, 0)),
            pl.BlockSpec((1, NODE_DIM), lambda i: (0, 0)),
        ],
        out_specs=pl.BlockSpec((BLK, NODE_DIM), lambda i: (i, 0)),
        out_shape=jax.ShapeDtypeStruct((N, NODE_DIM), jnp.float32),
        compiler_params=pltpu.CompilerParams(
            dimension_semantics=("arbitrary",)),
    )(x, s, cnt, batch2d, u, w1x, w1a, w1u, b1r, w2, b2r)


def kernel(x, edge_index, edge_attr, u, batch, W1, b1, W2, b2):
    col = edge_index[1].astype(jnp.int32)
    col2d = jnp.concatenate(
        [col, jnp.zeros(((COL_ROWS - N_CHUNKS) * CHUNK,), jnp.int32)]
    ).reshape(COL_ROWS, CHUNK)
    sums, cnt = _get_sc_scatter()(edge_attr, col2d)
    sums_bf = sums.astype(jnp.bfloat16)

    batch2d = batch.astype(jnp.int8).reshape(N, 1)
    w1x = W1[:NODE_DIM].astype(jnp.bfloat16)
    w1a = W1[NODE_DIM:NODE_DIM + EDGE_DIM].astype(jnp.bfloat16)
    w1u = W1[NODE_DIM + EDGE_DIM:]
    b1r = b1.reshape(1, HIDDEN_DIM)
    b2r = b2.reshape(1, NODE_DIM)
    w2_bf = W2.astype(jnp.bfloat16)
    return _tc_mlp(x, sums_bf, cnt, batch2d, u, w1x, w1a, w1u, b1r, w2_bf,
                   b2r)


# speedup vs baseline: 1.2398x; 1.0033x over previous
import functools

import jax
import jax.numpy as jnp
from jax import lax
from jax.experimental import pallas as pl
from jax.experimental.pallas import tpu as pltpu
from jax.experimental.pallas import tpu_sc as plsc

N = 10000
E = 160000
NODE_DIM = 256
EDGE_DIM = 16
GLOBAL_DIM = 64
HIDDEN_DIM = 512
B_GRAPHS = 64

N_TILES = 32
CHUNK = 128
N_CHUNKS = E // CHUNK
COL_ROWS = 1280
BASE_ROWS = N_CHUNKS // N_TILES
MAX_ROWS = BASE_ROWS + 1
ROWS_PER_TILE = 640
SUM_ROWS = 10240

BLK = 2560
GRID = (N + BLK - 1) // BLK


DEPTH = 8


def _sc_scatter_body(ea_hbm, col_hbm, sums_hbm, cnt_hbm,
                     edge_v, idx_v, cnt_v, zero_v, sums_sh, sem_in, sem_sc):
    cid = lax.axis_index("c")
    sid = lax.axis_index("s")
    w = cid * 16 + sid
    base = w * BASE_ROWS + jnp.minimum(w, 2)
    nrows = jnp.where(w < 2, MAX_ROWS, BASE_ROWS)
    dma_base = jnp.minimum(base, N_CHUNKS - MAX_ROWS)
    off = base - dma_base

    in0 = pltpu.async_copy(
        ea_hbm.at[pl.ds(dma_base * CHUNK, MAX_ROWS * CHUNK)], edge_v, sem_in)
    in1 = pltpu.async_copy(col_hbm.at[pl.ds(dma_base, MAX_ROWS)], idx_v,
                           sem_in)
    zeros16 = jnp.zeros((16,), jnp.float32)

    def zero_step(i, carry):
        cnt_v[pl.ds(i * 16, 16)] = zeros16
        zero_v[i] = zeros16
        return carry

    lax.fori_loop(0, ROWS_PER_TILE, zero_step, 0)
    pltpu.sync_copy(zero_v,
                    sums_sh.at[pl.ds(sid * ROWS_PER_TILE, ROWS_PER_TILE)])
    in0.wait()
    in1.wait()
    plsc.subcore_barrier()

    ones = jnp.full((16,), 1.0, jnp.float32)

    def scatter_of(j):
        row = off + j
        return pltpu.make_async_copy(edge_v.at[pl.ds(row * CHUNK, CHUNK)],
                                     sums_sh.at[idx_v.at[row]], sem_sc)

    def chunk_step(j, carry):
        row = off + j
        scatter_of(j).start(add=True)

        @pl.when(j >= DEPTH)
        def _():
            scatter_of(j - DEPTH).wait()

        for k in range(CHUNK // 16):
            c = idx_v[row, pl.ds(k * 16, 16)]
            plsc.addupdate_scatter(cnt_v, [c], ones)
        return carry

    lax.fori_loop(0, nrows, chunk_step, 0)

    def drain_step(j, carry):
        scatter_of(j).wait()
        return carry

    lax.fori_loop(nrows - DEPTH, nrows, drain_step, 0)
    plsc.subcore_barrier()

    pltpu.sync_copy(sums_sh.at[pl.ds(sid * ROWS_PER_TILE, ROWS_PER_TILE)],
                    sums_hbm.at[cid, pl.ds(sid * ROWS_PER_TILE,
                                           ROWS_PER_TILE)])
    pltpu.sync_copy(cnt_v, cnt_hbm.at[w])


@functools.cache
def _get_sc_scatter():
    return functools.partial(
        pl.kernel,
        out_type=[
            jax.ShapeDtypeStruct((2, SUM_ROWS, EDGE_DIM), jnp.float32),
            jax.ShapeDtypeStruct((N_TILES, SUM_ROWS), jnp.float32),
        ],
        mesh=plsc.VectorSubcoreMesh(core_axis_name="c", subcore_axis_name="s",
                                    num_cores=2, num_subcores=16),
        scratch_types=[
            pltpu.VMEM((MAX_ROWS * CHUNK, EDGE_DIM), jnp.float32),
            pltpu.VMEM((MAX_ROWS, CHUNK), jnp.int32),
            pltpu.VMEM((SUM_ROWS,), jnp.float32),
            pltpu.VMEM((ROWS_PER_TILE, EDGE_DIM), jnp.float32),
            pltpu.VMEM_SHARED((SUM_ROWS, EDGE_DIM), jnp.float32),
            pltpu.SemaphoreType.DMA,
            pltpu.SemaphoreType.DMA,
        ],
        compiler_params=pltpu.CompilerParams(needs_layout_passes=False,
                                             use_tc_tiling_on_sc=False),
    )(_sc_scatter_body)


def _tc_mlp_body(x_ref, s_ref, cnt_ref, b_ref, u_ref,
                 w1x_ref, w1a_ref, w1u_ref, b1_ref, w2_ref, b2_ref, o_ref):
    ones = jnp.ones((N_TILES, 1), jnp.float32)
    c = lax.dot_general(cnt_ref[...], ones, (((0,), (0,)), ((), ())),
                        preferred_element_type=jnp.float32)
    inv = 1.0 / jnp.maximum(c, 1.0)
    oh = (b_ref[...].astype(jnp.int32)
          == lax.broadcasted_iota(jnp.int32, (BLK, B_GRAPHS), 1)
          ).astype(jnp.float32)
    uw = jnp.dot(u_ref[...], w1u_ref[...], preferred_element_type=jnp.float32)
    acc = jnp.dot(x_ref[...].astype(jnp.bfloat16), w1x_ref[...],
                  preferred_element_type=jnp.float32)
    acc = acc + jnp.dot(s_ref[0] + s_ref[1], w1a_ref[...],
                        preferred_element_type=jnp.float32) * inv
    acc = acc + jnp.dot(oh, uw, preferred_element_type=jnp.float32)
    h1 = jnp.maximum(acc + b1_ref[...], 0.0)
    o_ref[...] = (jnp.dot(h1.astype(jnp.bfloat16), w2_ref[...],
                          preferred_element_type=jnp.float32) + b2_ref[...])


def _tc_mlp(x, s, cnt, batch2d, u, w1x, w1a, w1u, b1r, w2, b2r):
    return pl.pallas_call(
        _tc_mlp_body,
        grid=(GRID,),
        in_specs=[
            pl.BlockSpec((BLK, NODE_DIM), lambda i: (i, 0)),
            pl.BlockSpec((2, BLK, EDGE_DIM), lambda i: (0, i, 0)),
            pl.BlockSpec((N_TILES, BLK), lambda i: (0, i)),
            pl.BlockSpec((BLK, 1), lambda i: (i, 0)),
            pl.BlockSpec((B_GRAPHS, GLOBAL_DIM), lambda i: (0, 0)),
            pl.BlockSpec((NODE_DIM, HIDDEN_DIM), lambda i: (0, 0)),
            pl.BlockSpec((EDGE_DIM, HIDDEN_DIM), lambda i: (0, 0)),
            pl.BlockSpec((GLOBAL_DIM, HIDDEN_DIM), lambda i: (0, 0)),
            pl.BlockSpec((1, HIDDEN_DIM), lambda i: (0, 0)),
            pl.BlockSpec((HIDDEN_DIM, NODE_DIM), lambda i: (0, 0)),
            pl.BlockSpec((1, NODE_DIM), lambda i: (0, 0)),
        ],
        out_specs=pl.BlockSpec((BLK, NODE_DIM), lambda i: (i, 0)),
        out_shape=jax.ShapeDtypeStruct((N, NODE_DIM), jnp.float32),
        compiler_params=pltpu.CompilerParams(
            dimension_semantics=("arbitrary",)),
    )(x, s, cnt, batch2d, u, w1x, w1a, w1u, b1r, w2, b2r)


def kernel(x, edge_index, edge_attr, u, batch, W1, b1, W2, b2):
    col = edge_index[1].astype(jnp.int32)
    col2d = jnp.concatenate(
        [col, jnp.zeros(((COL_ROWS - N_CHUNKS) * CHUNK,), jnp.int32)]
    ).reshape(COL_ROWS, CHUNK)
    sums, cnt = _get_sc_scatter()(edge_attr, col2d)
    sums_bf = sums.astype(jnp.bfloat16)

    batch2d = batch.astype(jnp.int8).reshape(N, 1)
    w1x = W1[:NODE_DIM].astype(jnp.bfloat16)
    w1a = W1[NODE_DIM:NODE_DIM + EDGE_DIM].astype(jnp.bfloat16)
    w1u = W1[NODE_DIM + EDGE_DIM:]
    b1r = b1.reshape(1, HIDDEN_DIM)
    b2r = b2.reshape(1, NODE_DIM)
    w2_bf = W2.astype(jnp.bfloat16)
    return _tc_mlp(x, sums_bf, cnt, batch2d, u, w1x, w1a, w1u, b1r, w2_bf,
                   b2r)
